# Initial kernel scaffold; baseline (speedup 1.0000x reference)
import jax
import jax.numpy as jnp
from jax.experimental import pallas as pl


def kernel(x, batch):
    raise NotImplementedError("write your pallas kernel here")


# speedup vs baseline: 6.6401x; 6.6401x over previous
import functools

import jax
import jax.numpy as jnp
from jax import lax
from jax.experimental import pallas as pl
from jax.experimental.pallas import tpu as pltpu
from jax.experimental.pallas import tpu_sc as plsc

N = 320000
D = 128
S = 512
NC = 2
NS = 16
NW = NC * NS
R = N // NW
B = 80
NB = R // B
L = 16
KD = D // L


def _phase1_body(x_hbm, batch_hbm, partial_hbm, idx_v, buf0, buf1, acc, sem0, sem1):
  wid = lax.axis_index("s") * NC + lax.axis_index("c")
  r0 = wid * R
  minus_inf = jnp.full((L,), -jnp.inf, jnp.float32)

  pltpu.sync_copy(batch_hbm.at[pl.ds(r0, R)], idx_v)

  def init_body(i, _):
    for k in range(KD):
      acc[pl.ds(i * D + k * L, L)] = minus_inf
    return 0
  lax.fori_loop(0, S, init_body, 0)

  pltpu.async_copy(x_hbm.at[pl.ds(r0 * D, B * D)], buf0, sem0)
  pltpu.async_copy(x_hbm.at[pl.ds((r0 + B) * D, B * D)], buf1, sem1)

  def block_rows(buf, base_r, carry):
    def group_body(g, carry):
      ids16 = idx_v[pl.ds(base_r + g * L, L)]
      for i in range(L):
        prev = carry[0]
        cs = carry[1:]
        s = ids16[i]
        changed = s != prev

        @pl.when(changed)
        def _flush(cs=cs, prev=prev):
          for k in range(KD):
            acc[pl.ds(prev * D + k * L, L)] = cs[k]

        rb = (g * L + i) * D
        carry = (s,) + tuple(
            jnp.maximum(jnp.where(changed, minus_inf, cs[k]),
                        buf[pl.ds(rb + k * L, L)])
            for k in range(KD))
      return carry
    return lax.fori_loop(0, B // L, group_body, carry)

  def super_body(j, carry):
    for sub, (buf, sem) in enumerate(((buf0, sem0), (buf1, sem1))):
      b = 2 * j + sub
      pltpu.make_async_copy(x_hbm.at[pl.ds(0, B * D)], buf, sem).wait()
      carry = block_rows(buf, b * B, carry)
      nxt = jnp.minimum(b + 2, NB - 1)
      pltpu.async_copy(x_hbm.at[pl.ds((r0 + nxt * B) * D, B * D)], buf, sem)
    return carry

  carry0 = (idx_v[pl.ds(0, L)][0],) + (minus_inf,) * KD
  carry = lax.fori_loop(0, NB // 2, super_body, carry0)

  pltpu.make_async_copy(x_hbm.at[pl.ds(0, B * D)], buf0, sem0).wait()
  carry = block_rows(buf0, (NB - 1) * B, carry)
  pltpu.make_async_copy(x_hbm.at[pl.ds(0, B * D)], buf1, sem1).wait()

  prev = carry[0]
  for k in range(KD):
    acc[pl.ds(prev * D + k * L, L)] = carry[1 + k]

  pltpu.sync_copy(acc, partial_hbm.at[wid])


_phase1 = functools.partial(
    pl.kernel,
    out_type=jax.ShapeDtypeStruct((NW, S * D), jnp.float32),
    mesh=plsc.VectorSubcoreMesh(core_axis_name="c", subcore_axis_name="s"),
    scratch_types=[
        pltpu.VMEM((R,), jnp.int32),
        pltpu.VMEM((B * D,), jnp.float32),
        pltpu.VMEM((B * D,), jnp.float32),
        pltpu.VMEM((S * D,), jnp.float32),
        pltpu.SemaphoreType.DMA,
        pltpu.SemaphoreType.DMA,
    ],
)(_phase1_body)


def _merge_body(p_ref, o_ref):
  o_ref[...] = jnp.max(p_ref[...], axis=0)


def _phase2(partial):
  blk = S // 8
  return pl.pallas_call(
      _merge_body,
      out_shape=jax.ShapeDtypeStruct((S, D), jnp.float32),
      grid=(8,),
      in_specs=[pl.BlockSpec((NW, blk, D), lambda i: (0, i, 0))],
      out_specs=pl.BlockSpec((blk, D), lambda i: (i, 0)),
  )(partial)


@jax.jit
def kernel(x, batch):
  partial = _phase1(jnp.reshape(x, (N * D,)), batch)
  return _phase2(jnp.reshape(partial, (NW, S, D)))


# speedup vs baseline: 6.7512x; 1.0167x over previous
import functools

import jax
import jax.numpy as jnp
from jax import lax
from jax.experimental import pallas as pl
from jax.experimental.pallas import tpu as pltpu
from jax.experimental.pallas import tpu_sc as plsc

N = 320000
D = 128
S = 512
NC = 2
NS = 16
NW = NC * NS
R = N // NW
B = 80
NB = R // B
L = 16
KD = D // L


def _phase1_body(x_hbm, batch_hbm, partial_hbm,
                 idx_v, buf0, buf1, acc, cvec, sem0, sem1):
  wid = lax.axis_index("s") * NC + lax.axis_index("c")
  r0 = wid * R
  minus_inf = jnp.full((L,), -jnp.inf, jnp.float32)

  pltpu.sync_copy(batch_hbm.at[pl.ds(r0, R)], idx_v)

  def init_body(i, _):
    for k in range(KD):
      acc[pl.ds(i * D + k * L, L)] = minus_inf
    return 0
  lax.fori_loop(0, S, init_body, 0)
  for k in range(KD):
    cvec[pl.ds(k * L, L)] = minus_inf

  pltpu.async_copy(x_hbm.at[pl.ds(r0 * D, B * D)], buf0, sem0)
  pltpu.async_copy(x_hbm.at[pl.ds((r0 + B) * D, B * D)], buf1, sem1)

  def block_rows(buf, base_r, prev):
    def group_body(g, prev):
      ids16 = idx_v[pl.ds(base_r + g * L, L)]
      last = ids16[L - 1]
      uniform = last == prev

      @pl.when(uniform)
      def _fast():
        rb = g * L * D
        for k in range(KD):
          m = buf[pl.ds(rb + k * L, L)]
          for i in range(1, L):
            m = jnp.maximum(m, buf[pl.ds(rb + i * D + k * L, L)])
          cvec[pl.ds(k * L, L)] = jnp.maximum(cvec[pl.ds(k * L, L)], m)

      @pl.when(jnp.logical_not(uniform))
      def _slow():
        sprev = prev
        for i in range(L):
          s = ids16[i]
          changed = s != sprev

          @pl.when(changed)
          def _flush(sprev=sprev):
            for k in range(KD):
              acc[pl.ds(sprev * D + k * L, L)] = cvec[pl.ds(k * L, L)]
              cvec[pl.ds(k * L, L)] = minus_inf

          rb = (g * L + i) * D
          for k in range(KD):
            cvec[pl.ds(k * L, L)] = jnp.maximum(cvec[pl.ds(k * L, L)],
                                                buf[pl.ds(rb + k * L, L)])
          sprev = s

      return last
    return lax.fori_loop(0, B // L, group_body, prev)

  def super_body(j, prev):
    for sub, (buf, sem) in enumerate(((buf0, sem0), (buf1, sem1))):
      b = 2 * j + sub
      pltpu.make_async_copy(x_hbm.at[pl.ds(0, B * D)], buf, sem).wait()
      prev = block_rows(buf, b * B, prev)
      nxt = jnp.minimum(b + 2, NB - 1)
      pltpu.async_copy(x_hbm.at[pl.ds((r0 + nxt * B) * D, B * D)], buf, sem)
    return prev

  prev = lax.fori_loop(0, NB // 2, super_body, idx_v[pl.ds(0, L)][0])

  pltpu.make_async_copy(x_hbm.at[pl.ds(0, B * D)], buf0, sem0).wait()
  prev = block_rows(buf0, (NB - 1) * B, prev)
  pltpu.make_async_copy(x_hbm.at[pl.ds(0, B * D)], buf1, sem1).wait()

  for k in range(KD):
    acc[pl.ds(prev * D + k * L, L)] = cvec[pl.ds(k * L, L)]

  pltpu.sync_copy(acc, partial_hbm.at[wid])


_phase1 = functools.partial(
    pl.kernel,
    out_type=jax.ShapeDtypeStruct((NW, S * D), jnp.float32),
    mesh=plsc.VectorSubcoreMesh(core_axis_name="c", subcore_axis_name="s"),
    scratch_types=[
        pltpu.VMEM((R,), jnp.int32),
        pltpu.VMEM((B * D,), jnp.float32),
        pltpu.VMEM((B * D,), jnp.float32),
        pltpu.VMEM((S * D,), jnp.float32),
        pltpu.VMEM((D,), jnp.float32),
        pltpu.SemaphoreType.DMA,
        pltpu.SemaphoreType.DMA,
    ],
)(_phase1_body)


def _merge_body(p_ref, o_ref):
  o_ref[...] = jnp.max(p_ref[...], axis=0)


def _phase2(partial):
  blk = (S * D) // 8
  return pl.pallas_call(
      _merge_body,
      out_shape=jax.ShapeDtypeStruct((S * D,), jnp.float32),
      grid=(8,),
      in_specs=[pl.BlockSpec((NW, blk), lambda i: (0, i))],
      out_specs=pl.BlockSpec((blk,), lambda i: (i,)),
  )(partial)


@jax.jit
def kernel(x, batch):
  partial = _phase1(jnp.reshape(x, (N * D,)), batch)
  return jnp.reshape(_phase2(partial), (S, D))
